# Initial kernel scaffold; baseline (speedup 1.0000x reference)
import jax
import jax.numpy as jnp
from jax.experimental import pallas as pl


def kernel(inputs, edge_index, W, attn_l, attn_r):
    raise NotImplementedError("write your pallas kernel here")


# speedup vs baseline: 26.4168x; 26.4168x over previous
import jax
import jax.numpy as jnp
from jax import lax
from jax.experimental import pallas as pl
from jax.experimental.pallas import tpu as pltpu
from jax.experimental.pallas import tpu_sc as plsc

N = 10000
E = 320000
IN_DIM = 128
H = 8
D = 16
HD = H * D
ALPHA = 0.2

BR = 1000
GRID = N // BR
CH = 80
EPT = E // 16
NIT = EPT // CH
TW = 128
ACC_W = 128
ZR = 80
RB = 640


def _pre_body(x_ref, wt_ref, a1w_ref, a2w_ref, t_ref, m_ref):
    i = pl.program_id(0)
    ft = jnp.dot(x_ref[...], wt_ref[...], preferred_element_type=jnp.float32)
    a1 = jnp.dot(ft, a1w_ref[...], preferred_element_type=jnp.float32)
    a2 = jnp.dot(ft, a2w_ref[...], preferred_element_type=jnp.float32)
    pad = jnp.zeros((BR, 32), jnp.float32)
    t_ref[0] = jnp.concatenate([ft[:, :64], a1[:, :16], a2[:, :16], pad], axis=1)
    t_ref[1] = jnp.concatenate([ft[:, 64:], a1[:, :16], a2[:, :16], pad], axis=1)

    @pl.when(i == 0)
    def _():
        m_ref[...] = jnp.full((2, 128), -jnp.inf, jnp.float32)

    m1 = jnp.max(a1, axis=0, keepdims=True)
    m2 = jnp.max(a2, axis=0, keepdims=True)
    m_ref[0:1, :] = jnp.maximum(m_ref[0:1, :], m1)
    m_ref[1:2, :] = jnp.maximum(m_ref[1:2, :], m2)


_tc_pre = pl.pallas_call(
    _pre_body,
    grid=(GRID,),
    in_specs=[
        pl.BlockSpec((BR, IN_DIM), lambda i: (i, 0)),
        pl.BlockSpec((IN_DIM, HD), lambda i: (0, 0)),
        pl.BlockSpec((HD, 128), lambda i: (0, 0)),
        pl.BlockSpec((HD, 128), lambda i: (0, 0)),
    ],
    out_specs=[
        pl.BlockSpec((2, BR, TW), lambda i: (0, i, 0)),
        pl.BlockSpec((2, 128), lambda i: (0, 0)),
    ],
    out_shape=[
        jax.ShapeDtypeStruct((2, N, TW), jnp.float32),
        jax.ShapeDtypeStruct((2, 128), jnp.float32),
    ],
)


def _sc_body(t_hbm, m_hbm, srcb_hbm, dstb_hbm, dst_hbm, acc_out,
             srcv, dstbv, dstv, trows, a2rows, msg, zbuf, mbuf, acc, sem):
    c = lax.axis_index("c")
    s = lax.axis_index("s")
    iota = lax.broadcasted_iota(jnp.int32, (16,), 0)
    zeros16 = jnp.zeros((16,), jnp.float32)
    c4 = c * 4

    def _zrow(r, _):
        for k in range(ACC_W // 16):
            zbuf[r, pl.ds(k * 16, 16)] = zeros16
            msg[r, pl.ds(k * 16, 16)] = zeros16
        return 0
    lax.fori_loop(0, ZR, _zrow, 0)

    nch = jnp.where(s < 15, RB // ZR, (N - 15 * RB) // ZR)

    def _zacc(k, _):
        r0 = pl.multiple_of(s * RB + k * ZR, ZR)
        pltpu.sync_copy(zbuf, acc.at[pl.ds(r0, ZR), :])
        return 0
    lax.fori_loop(0, nch, _zacc, 0)
    plsc.subcore_barrier()

    pltpu.sync_copy(m_hbm, mbuf)
    msum = mbuf[0, pl.ds(0, 16)] + mbuf[1, pl.ds(0, 16)]
    m16 = jnp.maximum(msum, msum * ALPHA)

    def step(i, _):
        base = s * EPT + i * CH
        pltpu.sync_copy(srcb_hbm.at[pl.ds(c * E + base, CH)], srcv)
        pltpu.sync_copy(dstb_hbm.at[pl.ds(c * E + base, CH)], dstbv)
        pltpu.sync_copy(dst_hbm.at[pl.ds(base, CH)], dstv)
        cps = pltpu.async_copy(t_hbm.at[srcv], trows, sem)
        cpd = pltpu.async_copy(t_hbm.at[dstbv], a2rows, sem)
        cps.wait()
        cpd.wait()

        def edge(e, _):
            se = jnp.broadcast_to(e, (16,))
            va1 = trows[e, pl.ds(64, 16)]
            va2 = a2rows[e, pl.ds(80, 16)]
            t = va1 + va2
            logit = jnp.maximum(t, t * ALPHA)
            p = jnp.exp(logit - m16)
            for j in range(4):
                ftb = trows[e, pl.ds(j * 16, 16)]
                q = jnp.take_along_axis(
                    p, jnp.broadcast_to(c4 + j, (16,)), axis=0)
                msg[e, pl.ds(j * 16, 16)] = ftb * q
            zmask = (iota >= c4) & (iota < c4 + 4)
            plsc.store_scatter(msg, [se, iota + (64 - c4)], p, mask=zmask)
            return 0
        lax.fori_loop(0, CH, edge, 0)

        pltpu.sync_copy(msg, acc.at[dstv], add=True)
        return 0
    lax.fori_loop(0, NIT, step, 0)

    plsc.subcore_barrier()

    def _wout(k, _):
        r0 = pl.multiple_of(s * RB + k * ZR, ZR)
        pltpu.sync_copy(acc.at[pl.ds(r0, ZR), :],
                        acc_out.at[pl.ds(c * N + r0, ZR), :])
        return 0
    lax.fori_loop(0, nch, _wout, 0)


_sc_edge = pl.kernel(
    _sc_body,
    out_type=jax.ShapeDtypeStruct((2 * N, ACC_W), jnp.float32),
    mesh=plsc.VectorSubcoreMesh(core_axis_name="c", subcore_axis_name="s"),
    compiler_params=pltpu.CompilerParams(needs_layout_passes=False),
    scratch_types=[
        pltpu.VMEM((CH,), jnp.int32),
        pltpu.VMEM((CH,), jnp.int32),
        pltpu.VMEM((CH,), jnp.int32),
        pltpu.VMEM((CH, TW), jnp.float32),
        pltpu.VMEM((CH, TW), jnp.float32),
        pltpu.VMEM((CH, ACC_W), jnp.float32),
        pltpu.VMEM((ZR, ACC_W), jnp.float32),
        pltpu.VMEM((2, 128), jnp.float32),
        pltpu.VMEM_SHARED((N, ACC_W), jnp.float32),
        pltpu.SemaphoreType.DMA,
    ],
)


def _post_body(acc0_ref, acc1_ref, s_ref, out_ref):
    sel = s_ref[...]
    n0 = acc0_ref[:, 0:64]
    z0 = acc0_ref[:, 64:68]
    n1 = acc1_ref[:, 0:64]
    z1 = acc1_ref[:, 64:68]
    r0 = n0 / jnp.dot(z0, sel, preferred_element_type=jnp.float32)
    r1 = n1 / jnp.dot(z1, sel, preferred_element_type=jnp.float32)
    out_ref[...] = jnp.concatenate([r0, r1], axis=1)


_tc_post = pl.pallas_call(
    _post_body,
    grid=(GRID,),
    in_specs=[
        pl.BlockSpec((BR, ACC_W), lambda i: (i, 0)),
        pl.BlockSpec((BR, ACC_W), lambda i: (N // BR + i, 0)),
        pl.BlockSpec((4, 64), lambda i: (0, 0)),
    ],
    out_specs=pl.BlockSpec((BR, HD), lambda i: (i, 0)),
    out_shape=jax.ShapeDtypeStruct((N, HD), jnp.float32),
)


@jax.jit
def kernel(inputs, edge_index, W, attn_l, attn_r):
    wt = W.T
    al = attn_l[:, :, 0]
    ar = attn_r[:, :, 0]
    eye = jnp.eye(H, dtype=jnp.float32)
    a1w = jnp.einsum('hd,hk->hdk', al, eye).reshape(HD, H)
    a2w = jnp.einsum('hd,hk->hdk', ar, eye).reshape(HD, H)
    zpad = jnp.zeros((HD, 128 - 2 * H), jnp.float32)
    a1w = jnp.concatenate([a1w, a1w, zpad], axis=1)
    a2w = jnp.concatenate([a2w, a2w, zpad], axis=1)
    sel = jnp.einsum('jk,d->kjd', jnp.eye(4, dtype=jnp.float32),
                     jnp.ones((D,), jnp.float32)).reshape(4, 64)

    tpack, m = _tc_pre(inputs, wt, a1w, a2w)
    src = edge_index[0]
    dst = edge_index[1]
    srcb = jnp.concatenate([src, src + N])
    dstb = jnp.concatenate([dst, dst + N])
    acc = _sc_edge(tpack.reshape(2 * N, TW), m, srcb, dstb, dst)
    out = _tc_post(acc, acc, sel)
    return out.reshape(N, H, D)


# speedup vs baseline: 31.9917x; 1.2110x over previous
import jax
import jax.numpy as jnp
from jax import lax
from jax.experimental import pallas as pl
from jax.experimental.pallas import tpu as pltpu
from jax.experimental.pallas import tpu_sc as plsc

N = 10000
E = 320000
IN_DIM = 128
H = 8
D = 16
HD = H * D
ALPHA = 0.2

BR = 1000
GRID = N // BR
CH = 32
NIT = 640
EP = NIT * CH
EPAD = 16 * EP
PAD = EPAD - E
SDW = 2 * CH
TW = 128
ACC_W = 128
ZR = 16
RB = 624
DUMMY = 2 * N


def _pre_body(x_ref, wt_ref, a1w_ref, a2w_ref, t_ref, m_ref):
    i = pl.program_id(0)
    ft = jnp.dot(x_ref[...], wt_ref[...], preferred_element_type=jnp.float32)
    a1 = jnp.dot(ft, a1w_ref[...], preferred_element_type=jnp.float32)
    a2 = jnp.dot(ft, a2w_ref[...], preferred_element_type=jnp.float32)
    pad = jnp.zeros((BR, 32), jnp.float32)
    t_ref[0] = jnp.concatenate([ft[:, :64], a1[:, :16], a2[:, :16], pad], axis=1)
    t_ref[1] = jnp.concatenate([ft[:, 64:], a1[:, :16], a2[:, :16], pad], axis=1)

    @pl.when(i == 0)
    def _():
        m_ref[...] = jnp.full((2, 128), -jnp.inf, jnp.float32)

    m1 = jnp.max(a1, axis=0, keepdims=True)
    m2 = jnp.max(a2, axis=0, keepdims=True)
    m_ref[0:1, :] = jnp.maximum(m_ref[0:1, :], m1)
    m_ref[1:2, :] = jnp.maximum(m_ref[1:2, :], m2)


_tc_pre = pl.pallas_call(
    _pre_body,
    grid=(GRID,),
    in_specs=[
        pl.BlockSpec((BR, IN_DIM), lambda i: (i, 0)),
        pl.BlockSpec((IN_DIM, HD), lambda i: (0, 0)),
        pl.BlockSpec((HD, 128), lambda i: (0, 0)),
        pl.BlockSpec((HD, 128), lambda i: (0, 0)),
    ],
    out_specs=[
        pl.BlockSpec((2, BR, TW), lambda i: (0, i, 0)),
        pl.BlockSpec((2, 128), lambda i: (0, 0)),
    ],
    out_shape=[
        jax.ShapeDtypeStruct((2, N, TW), jnp.float32),
        jax.ShapeDtypeStruct((2, 128), jnp.float32),
    ],
)


def _sc_body(t_hbm, m_hbm, sd_hbm, acc_out,
             sdrow, tr0, tr1, a2r0, a2r1, msg0, msg1, dsv, zbuf, mbuf, acc,
             gsem0, gsem1, ssem0, ssem1, isem0, isem1, isem2, isem3):
    c = lax.axis_index("c")
    s = lax.axis_index("s")
    iota = lax.broadcasted_iota(jnp.int32, (16,), 0)
    zeros16 = jnp.zeros((16,), jnp.float32)
    c4 = c * 4

    trows = (tr0, tr1)
    a2rows = (a2r0, a2r1)
    msgs = (msg0, msg1)
    gsems = (gsem0, gsem1)
    ssems = (ssem0, ssem1)
    isems = (isem0, isem1, isem2, isem3)

    def _zrow(r, _):
        for k in range(ACC_W // 16):
            zbuf[r, pl.ds(k * 16, 16)] = zeros16
            msg0[r, pl.ds(k * 16, 16)] = zeros16
            msg1[r, pl.ds(k * 16, 16)] = zeros16
        return 0
    lax.fori_loop(0, ZR, _zrow, 0)

    def _zrow2(r, _):
        for k in range(ACC_W // 16):
            msg0[r, pl.ds(k * 16, 16)] = zeros16
            msg1[r, pl.ds(k * 16, 16)] = zeros16
        return 0
    lax.fori_loop(ZR, CH, _zrow2, 0)

    nch = jnp.where(s < 15, RB // ZR, (N - 15 * RB) // ZR)

    def _zacc(k, _):
        r0 = pl.multiple_of(s * RB + k * ZR, ZR)
        pltpu.sync_copy(zbuf, acc.at[pl.ds(r0, ZR), :])
        return 0
    lax.fori_loop(0, nch, _zacc, 0)

    pltpu.sync_copy(m_hbm, mbuf)
    msum = mbuf[0, pl.ds(0, 16)] + mbuf[1, pl.ds(0, 16)]
    m16 = jnp.maximum(msum, msum * ALPHA)
    plsc.subcore_barrier()

    sd0 = (c * 16 + s) * (NIT * SDW)

    def idx_copy(i, b4):
        pltpu.async_copy(sd_hbm.at[pl.ds(sd0 + i * SDW, SDW)],
                         sdrow.at[pl.ds(b4 * SDW, SDW)], isems[b4])

    def idx_wait(i, b4):
        pltpu.make_async_copy(sd_hbm.at[pl.ds(sd0 + i * SDW, SDW)],
                              sdrow.at[pl.ds(b4 * SDW, SDW)], isems[b4]).wait()

    def start_gathers(i, sb, b4):
        pltpu.async_copy(
            t_hbm.at[sdrow.at[pl.ds(b4 * SDW, CH)]], trows[sb], gsems[sb])
        pltpu.async_copy(
            t_hbm.at[sdrow.at[pl.ds(b4 * SDW + CH, CH)]], a2rows[sb],
            gsems[sb])

    def wait_gathers(i, sb, b4):
        pltpu.make_async_copy(
            t_hbm.at[sdrow.at[pl.ds(b4 * SDW, CH)]], trows[sb],
            gsems[sb]).wait()
        pltpu.make_async_copy(
            t_hbm.at[sdrow.at[pl.ds(b4 * SDW + CH, CH)]], a2rows[sb],
            gsems[sb]).wait()

    def wait_scatter(sb, b4):
        pltpu.make_async_copy(
            msgs[sb], acc.at[dsv.at[b4]], ssems[sb]).wait()

    for k in range(4):
        idx_copy(k, k)
    idx_wait(0, 0)
    idx_wait(1, 1)
    start_gathers(0, 0, 0)
    start_gathers(1, 1, 1)

    def step(i4, _):
        for b in range(4):
            sb = b % 2
            i = i4 * 4 + b
            wait_gathers(i, sb, b)
            if b < 2:
                @pl.when(i4 >= 1)
                def _():
                    wait_scatter(sb, (b + 2) % 4)
            else:
                wait_scatter(sb, (b + 2) % 4)

            for k in range(CH // 16):
                dsv[b, pl.ds(k * 16, 16)] = \
                    sdrow[pl.ds(b * SDW + CH + k * 16, 16)]

            tr = trows[sb]
            a2r = a2rows[sb]
            msg = msgs[sb]

            def edge(e, _):
                se = jnp.broadcast_to(e, (16,))
                va1 = tr[e, pl.ds(64, 16)]
                va2 = a2r[e, pl.ds(80, 16)]
                t = va1 + va2
                logit = jnp.maximum(t, t * ALPHA)
                p = jnp.exp(logit - m16)
                for j in range(4):
                    ftb = tr[e, pl.ds(j * 16, 16)]
                    q = jnp.take_along_axis(
                        p, jnp.broadcast_to(c4 + j, (16,)), axis=0)
                    msg[e, pl.ds(j * 16, 16)] = ftb * q
                zmask = (iota >= c4) & (iota < c4 + 4)
                plsc.store_scatter(msg, [se, iota + (64 - c4)], p, mask=zmask)
                return 0
            lax.fori_loop(0, CH, edge, 0)

            pltpu.async_copy(msg, acc.at[dsv.at[b]], ssems[sb], add=True)

            nb = (b + 2) % 4
            if b < 2:
                idx_wait(i + 2, nb)
                start_gathers(i + 2, sb, nb)
            else:
                @pl.when(i4 < NIT // 4 - 1)
                def _():
                    idx_wait(i + 2, nb)
                    start_gathers(i + 2, sb, nb)

            @pl.when(i4 < NIT // 4 - 1)
            def _():
                idx_copy(i + 4, b)
        return 0
    lax.fori_loop(0, NIT // 4, step, 0)

    wait_scatter(0, 2)
    wait_scatter(1, 3)
    plsc.subcore_barrier()

    def _wout(k, _):
        r0 = pl.multiple_of(s * RB + k * ZR, ZR)
        pltpu.sync_copy(acc.at[pl.ds(r0, ZR), :],
                        acc_out.at[pl.ds(c * N + r0, ZR), :])
        return 0
    lax.fori_loop(0, nch, _wout, 0)


_sc_edge = pl.kernel(
    _sc_body,
    out_type=jax.ShapeDtypeStruct((2 * N, ACC_W), jnp.float32),
    mesh=plsc.VectorSubcoreMesh(core_axis_name="c", subcore_axis_name="s"),
    compiler_params=pltpu.CompilerParams(needs_layout_passes=False),
    scratch_types=[
        pltpu.VMEM((4 * SDW,), jnp.int32),
        pltpu.VMEM((CH, TW), jnp.float32),
        pltpu.VMEM((CH, TW), jnp.float32),
        pltpu.VMEM((CH, TW), jnp.float32),
        pltpu.VMEM((CH, TW), jnp.float32),
        pltpu.VMEM((CH, ACC_W), jnp.float32),
        pltpu.VMEM((CH, ACC_W), jnp.float32),
        pltpu.VMEM((4, CH), jnp.int32),
        pltpu.VMEM((ZR, ACC_W), jnp.float32),
        pltpu.VMEM((2, 128), jnp.float32),
        pltpu.VMEM_SHARED((N, ACC_W), jnp.float32),
        pltpu.SemaphoreType.DMA,
        pltpu.SemaphoreType.DMA,
        pltpu.SemaphoreType.DMA,
        pltpu.SemaphoreType.DMA,
        pltpu.SemaphoreType.DMA,
        pltpu.SemaphoreType.DMA,
        pltpu.SemaphoreType.DMA,
        pltpu.SemaphoreType.DMA,
    ],
)


def _post_body(acc0_ref, acc1_ref, s_ref, out_ref):
    sel = s_ref[...]
    n0 = acc0_ref[:, 0:64]
    z0 = acc0_ref[:, 64:68]
    n1 = acc1_ref[:, 0:64]
    z1 = acc1_ref[:, 64:68]
    r0 = n0 / jnp.dot(z0, sel, preferred_element_type=jnp.float32)
    r1 = n1 / jnp.dot(z1, sel, preferred_element_type=jnp.float32)
    out_ref[...] = jnp.concatenate([r0, r1], axis=1)


_tc_post = pl.pallas_call(
    _post_body,
    grid=(GRID,),
    in_specs=[
        pl.BlockSpec((BR, ACC_W), lambda i: (i, 0)),
        pl.BlockSpec((BR, ACC_W), lambda i: (N // BR + i, 0)),
        pl.BlockSpec((4, 64), lambda i: (0, 0)),
    ],
    out_specs=pl.BlockSpec((BR, HD), lambda i: (i, 0)),
    out_shape=jax.ShapeDtypeStruct((N, HD), jnp.float32),
)


@jax.jit
def kernel(inputs, edge_index, W, attn_l, attn_r):
    wt = W.T
    al = attn_l[:, :, 0]
    ar = attn_r[:, :, 0]
    eye = jnp.eye(H, dtype=jnp.float32)
    a1w = jnp.einsum('hd,hk->hdk', al, eye).reshape(HD, H)
    a2w = jnp.einsum('hd,hk->hdk', ar, eye).reshape(HD, H)
    zpad = jnp.zeros((HD, 128 - 2 * H), jnp.float32)
    a1w = jnp.concatenate([a1w, a1w, zpad], axis=1)
    a2w = jnp.concatenate([a2w, a2w, zpad], axis=1)
    sel = jnp.einsum('jk,d->kjd', jnp.eye(4, dtype=jnp.float32),
                     jnp.ones((D,), jnp.float32)).reshape(4, 64)

    tpack, m = _tc_pre(inputs, wt, a1w, a2w)
    dummy = jnp.zeros((8, TW), jnp.float32).at[:, 64:80].set(-1e30)
    table = jnp.concatenate([tpack.reshape(2 * N, TW), dummy], axis=0)
    src = edge_index[0]
    dst = edge_index[1]
    spad = jnp.full((PAD,), DUMMY, jnp.int32)
    dpad = jnp.zeros((PAD,), jnp.int32)
    srcb = jnp.concatenate([src, spad, src + N, spad]).reshape(2, EPAD // CH, CH)
    dstp = jnp.concatenate([dst, dpad]).reshape(EPAD // CH, CH)
    dst2 = jnp.stack([dstp, dstp])
    sd = jnp.concatenate([srcb, dst2], axis=-1).reshape(-1)

    acc = _sc_edge(table, m, sd)
    out = _tc_post(acc, acc, sel)
    return out.reshape(N, H, D)


# speedup vs baseline: 36.9616x; 1.1553x over previous
import jax
import jax.numpy as jnp
from jax import lax
from jax.experimental import pallas as pl
from jax.experimental.pallas import tpu as pltpu
from jax.experimental.pallas import tpu_sc as plsc

N = 10000
E = 320000
IN_DIM = 128
H = 8
D = 16
HD = H * D
ALPHA = 0.2

BR = 1000
GRID = N // BR
CH = 32
NIT = 640
EP = NIT * CH
EPAD = 16 * EP
PAD = EPAD - E
SDW = 2 * CH
TW = 128
ACC_W = 128
ZR = 16
RB = 624
DUMMY = 2 * N


def _pre_body(x_ref, wt_ref, a1w_ref, a2w_ref, t_ref, m_ref):
    i = pl.program_id(0)
    ft = jnp.dot(x_ref[...], wt_ref[...], preferred_element_type=jnp.float32)
    a1 = jnp.dot(ft, a1w_ref[...], preferred_element_type=jnp.float32)
    a2 = jnp.dot(ft, a2w_ref[...], preferred_element_type=jnp.float32)
    pad = jnp.zeros((BR, 32), jnp.float32)
    t_ref[0] = jnp.concatenate([ft[:, :64], a1[:, :16], a2[:, :16], pad], axis=1)
    t_ref[1] = jnp.concatenate([ft[:, 64:], a1[:, :16], a2[:, :16], pad], axis=1)

    @pl.when(i == 0)
    def _():
        m_ref[...] = jnp.full((2, 128), -jnp.inf, jnp.float32)

    m1 = jnp.max(a1, axis=0, keepdims=True)
    m2 = jnp.max(a2, axis=0, keepdims=True)
    m_ref[0:1, :] = jnp.maximum(m_ref[0:1, :], m1)
    m_ref[1:2, :] = jnp.maximum(m_ref[1:2, :], m2)


_tc_pre = pl.pallas_call(
    _pre_body,
    grid=(GRID,),
    in_specs=[
        pl.BlockSpec((BR, IN_DIM), lambda i: (i, 0)),
        pl.BlockSpec((IN_DIM, HD), lambda i: (0, 0)),
        pl.BlockSpec((HD, 128), lambda i: (0, 0)),
        pl.BlockSpec((HD, 128), lambda i: (0, 0)),
    ],
    out_specs=[
        pl.BlockSpec((2, BR, TW), lambda i: (0, i, 0)),
        pl.BlockSpec((2, 128), lambda i: (0, 0)),
    ],
    out_shape=[
        jax.ShapeDtypeStruct((2, N, TW), jnp.float32),
        jax.ShapeDtypeStruct((2, 128), jnp.float32),
    ],
)


def _sc_body(t_hbm, m_hbm, sd_hbm, acc_out,
             sdrow, tr0, tr1, a2r0, a2r1, msg0, msg1, dsv, zbuf, mbuf, acc,
             gsem0, gsem1, ssem0, ssem1, isem0, isem1, isem2, isem3):
    c = lax.axis_index("c")
    s = lax.axis_index("s")
    iota = lax.broadcasted_iota(jnp.int32, (16,), 0)
    zeros16 = jnp.zeros((16,), jnp.float32)
    c4 = c * 4
    qidx = [jnp.broadcast_to(c4 + j, (16,)) for j in range(4)]
    rotidx = (iota + c4) & 15

    trows = (tr0, tr1)
    a2rows = (a2r0, a2r1)
    msgs = (msg0, msg1)
    gsems = (gsem0, gsem1)
    ssems = (ssem0, ssem1)
    isems = (isem0, isem1, isem2, isem3)

    def _zrow(r, _):
        for k in range(ACC_W // 16):
            zbuf[r, pl.ds(k * 16, 16)] = zeros16
            msg0[r, pl.ds(k * 16, 16)] = zeros16
            msg1[r, pl.ds(k * 16, 16)] = zeros16
        return 0
    lax.fori_loop(0, ZR, _zrow, 0)

    def _zrow2(r, _):
        for k in range(ACC_W // 16):
            msg0[r, pl.ds(k * 16, 16)] = zeros16
            msg1[r, pl.ds(k * 16, 16)] = zeros16
        return 0
    lax.fori_loop(ZR, CH, _zrow2, 0)

    nch = jnp.where(s < 15, RB // ZR, (N - 15 * RB) // ZR)

    def _zacc(k, _):
        r0 = pl.multiple_of(s * RB + k * ZR, ZR)
        pltpu.sync_copy(zbuf, acc.at[pl.ds(r0, ZR), :])
        return 0
    lax.fori_loop(0, nch, _zacc, 0)

    pltpu.sync_copy(m_hbm, mbuf)
    msum = mbuf[0, pl.ds(0, 16)] + mbuf[1, pl.ds(0, 16)]
    m16 = jnp.maximum(msum, msum * ALPHA)
    plsc.subcore_barrier()

    sd0 = (c * 16 + s) * (NIT * SDW)

    def idx_copy(i, b4):
        pltpu.async_copy(sd_hbm.at[pl.ds(sd0 + i * SDW, SDW)],
                         sdrow.at[pl.ds(b4 * SDW, SDW)], isems[b4])

    def idx_wait(i, b4):
        pltpu.make_async_copy(sd_hbm.at[pl.ds(sd0 + i * SDW, SDW)],
                              sdrow.at[pl.ds(b4 * SDW, SDW)], isems[b4]).wait()

    def start_gathers(i, sb, b4):
        pltpu.async_copy(
            t_hbm.at[sdrow.at[pl.ds(b4 * SDW, CH)]], trows[sb], gsems[sb])
        pltpu.async_copy(
            t_hbm.at[sdrow.at[pl.ds(b4 * SDW + CH, CH)]], a2rows[sb],
            gsems[sb])

    def wait_gathers(i, sb, b4):
        pltpu.make_async_copy(
            t_hbm.at[sdrow.at[pl.ds(b4 * SDW, CH)]], trows[sb],
            gsems[sb]).wait()
        pltpu.make_async_copy(
            t_hbm.at[sdrow.at[pl.ds(b4 * SDW + CH, CH)]], a2rows[sb],
            gsems[sb]).wait()

    def wait_scatter(sb, b4):
        pltpu.make_async_copy(
            msgs[sb], acc.at[dsv.at[b4]], ssems[sb]).wait()

    for k in range(4):
        idx_copy(k, k)
    idx_wait(0, 0)
    idx_wait(1, 1)
    start_gathers(0, 0, 0)
    start_gathers(1, 1, 1)

    def step(i4, _):
        for b in range(4):
            sb = b % 2
            i = i4 * 4 + b
            wait_gathers(i, sb, b)
            if b < 2:
                @pl.when(i4 >= 1)
                def _():
                    wait_scatter(sb, (b + 2) % 4)
            else:
                wait_scatter(sb, (b + 2) % 4)

            for k in range(CH // 16):
                dsv[b, pl.ds(k * 16, 16)] = \
                    sdrow[pl.ds(b * SDW + CH + k * 16, 16)]

            tr = trows[sb]
            a2r = a2rows[sb]
            msg = msgs[sb]

            @plsc.parallel_loop(0, CH, 1, unroll=4)
            def edge(e):
                va1 = tr[e, pl.ds(64, 16)]
                va2 = a2r[e, pl.ds(80, 16)]
                t = va1 + va2
                logit = jnp.maximum(t, t * ALPHA)
                p = jnp.exp(logit - m16)
                for j in range(4):
                    ftb = tr[e, pl.ds(j * 16, 16)]
                    q = jnp.take_along_axis(p, qidx[j], axis=0)
                    msg[e, pl.ds(j * 16, 16)] = ftb * q
                msg[e, pl.ds(64, 16)] = jnp.take_along_axis(p, rotidx, axis=0)

            pltpu.async_copy(msg, acc.at[dsv.at[b]], ssems[sb], add=True)

            nb = (b + 2) % 4
            if b < 2:
                idx_wait(i + 2, nb)
                start_gathers(i + 2, sb, nb)
            else:
                @pl.when(i4 < NIT // 4 - 1)
                def _():
                    idx_wait(i + 2, nb)
                    start_gathers(i + 2, sb, nb)

            @pl.when(i4 < NIT // 4 - 1)
            def _():
                idx_copy(i + 4, b)
        return 0
    lax.fori_loop(0, NIT // 4, step, 0)

    wait_scatter(0, 2)
    wait_scatter(1, 3)
    plsc.subcore_barrier()

    def _wout(k, _):
        r0 = pl.multiple_of(s * RB + k * ZR, ZR)
        pltpu.sync_copy(acc.at[pl.ds(r0, ZR), :],
                        acc_out.at[pl.ds(c * N + r0, ZR), :])
        return 0
    lax.fori_loop(0, nch, _wout, 0)


_sc_edge = pl.kernel(
    _sc_body,
    out_type=jax.ShapeDtypeStruct((2 * N, ACC_W), jnp.float32),
    mesh=plsc.VectorSubcoreMesh(core_axis_name="c", subcore_axis_name="s"),
    compiler_params=pltpu.CompilerParams(needs_layout_passes=False),
    scratch_types=[
        pltpu.VMEM((4 * SDW,), jnp.int32),
        pltpu.VMEM((CH, TW), jnp.float32),
        pltpu.VMEM((CH, TW), jnp.float32),
        pltpu.VMEM((CH, TW), jnp.float32),
        pltpu.VMEM((CH, TW), jnp.float32),
        pltpu.VMEM((CH, ACC_W), jnp.float32),
        pltpu.VMEM((CH, ACC_W), jnp.float32),
        pltpu.VMEM((4, CH), jnp.int32),
        pltpu.VMEM((ZR, ACC_W), jnp.float32),
        pltpu.VMEM((2, 128), jnp.float32),
        pltpu.VMEM_SHARED((N, ACC_W), jnp.float32),
        pltpu.SemaphoreType.DMA,
        pltpu.SemaphoreType.DMA,
        pltpu.SemaphoreType.DMA,
        pltpu.SemaphoreType.DMA,
        pltpu.SemaphoreType.DMA,
        pltpu.SemaphoreType.DMA,
        pltpu.SemaphoreType.DMA,
        pltpu.SemaphoreType.DMA,
    ],
)


def _post_body(acc0_ref, acc1_ref, s_ref, out_ref):
    sel = s_ref[...]
    n0 = acc0_ref[:, 0:64]
    z0 = acc0_ref[:, 64:68]
    n1 = acc1_ref[:, 0:64]
    z1 = acc1_ref[:, 64:68]
    r0 = n0 / jnp.dot(z0, sel, preferred_element_type=jnp.float32)
    r1 = n1 / jnp.dot(z1, sel, preferred_element_type=jnp.float32)
    out_ref[...] = jnp.concatenate([r0, r1], axis=1)


_tc_post = pl.pallas_call(
    _post_body,
    grid=(GRID,),
    in_specs=[
        pl.BlockSpec((BR, ACC_W), lambda i: (i, 0)),
        pl.BlockSpec((BR, ACC_W), lambda i: (N // BR + i, 0)),
        pl.BlockSpec((4, 64), lambda i: (0, 0)),
    ],
    out_specs=pl.BlockSpec((BR, HD), lambda i: (i, 0)),
    out_shape=jax.ShapeDtypeStruct((N, HD), jnp.float32),
)


@jax.jit
def kernel(inputs, edge_index, W, attn_l, attn_r):
    wt = W.T
    al = attn_l[:, :, 0]
    ar = attn_r[:, :, 0]
    eye = jnp.eye(H, dtype=jnp.float32)
    a1w = jnp.einsum('hd,hk->hdk', al, eye).reshape(HD, H)
    a2w = jnp.einsum('hd,hk->hdk', ar, eye).reshape(HD, H)
    zpad = jnp.zeros((HD, 128 - 2 * H), jnp.float32)
    a1w = jnp.concatenate([a1w, a1w, zpad], axis=1)
    a2w = jnp.concatenate([a2w, a2w, zpad], axis=1)
    sel = jnp.einsum('jk,d->kjd', jnp.eye(4, dtype=jnp.float32),
                     jnp.ones((D,), jnp.float32)).reshape(4, 64)

    tpack, m = _tc_pre(inputs, wt, a1w, a2w)
    dummy = jnp.zeros((8, TW), jnp.float32).at[:, 64:80].set(-1e30)
    table = jnp.concatenate([tpack.reshape(2 * N, TW), dummy], axis=0)
    src = edge_index[0]
    dst = edge_index[1]
    spad = jnp.full((PAD,), DUMMY, jnp.int32)
    dpad = jnp.zeros((PAD,), jnp.int32)
    srcb = jnp.concatenate([src, spad, src + N, spad]).reshape(2, EPAD // CH, CH)
    dstp = jnp.concatenate([dst, dpad]).reshape(EPAD // CH, CH)
    dst2 = jnp.stack([dstp, dstp])
    sd = jnp.concatenate([srcb, dst2], axis=-1).reshape(-1)

    acc = _sc_edge(table, m, sd)
    out = _tc_post(acc, acc, sel)
    return out.reshape(N, H, D)


# speedup vs baseline: 39.6646x; 1.0731x over previous
import jax
import jax.numpy as jnp
from jax import lax
from jax.experimental import pallas as pl
from jax.experimental.pallas import tpu as pltpu
from jax.experimental.pallas import tpu_sc as plsc

N = 10000
E = 320000
IN_DIM = 128
H = 8
D = 16
HD = H * D
ALPHA = 0.2

BR = 1000
GRID = N // BR
CH = 32
NIT = 640
EP = NIT * CH
EPAD = 16 * EP
PAD = EPAD - E
SDW = 2 * CH
TW = 128
ACC_W = 128
ZR = 16
RB = 624
DUMMY = 2 * N


def _pre_body(x_ref, wt_ref, a1w_ref, a2w_ref, t_ref, m_ref):
    i = pl.program_id(0)
    ft = jnp.dot(x_ref[...], wt_ref[...], preferred_element_type=jnp.float32)
    a1 = jnp.dot(ft, a1w_ref[...], preferred_element_type=jnp.float32)
    a2 = jnp.dot(ft, a2w_ref[...], preferred_element_type=jnp.float32)
    pad = jnp.zeros((BR, 32), jnp.float32)
    t_ref[0] = jnp.concatenate([ft[:, :64], a1[:, :16], a2[:, :16], pad], axis=1)
    t_ref[1] = jnp.concatenate([ft[:, 64:], a1[:, :16], a2[:, :16], pad], axis=1)

    @pl.when(i == 0)
    def _():
        m_ref[...] = jnp.full((2, 128), -jnp.inf, jnp.float32)

    m1 = jnp.max(a1, axis=0, keepdims=True)
    m2 = jnp.max(a2, axis=0, keepdims=True)
    m_ref[0:1, :] = jnp.maximum(m_ref[0:1, :], m1)
    m_ref[1:2, :] = jnp.maximum(m_ref[1:2, :], m2)


_tc_pre = pl.pallas_call(
    _pre_body,
    grid=(GRID,),
    in_specs=[
        pl.BlockSpec((BR, IN_DIM), lambda i: (i, 0)),
        pl.BlockSpec((IN_DIM, HD), lambda i: (0, 0)),
        pl.BlockSpec((HD, 128), lambda i: (0, 0)),
        pl.BlockSpec((HD, 128), lambda i: (0, 0)),
    ],
    out_specs=[
        pl.BlockSpec((2, BR, TW), lambda i: (0, i, 0)),
        pl.BlockSpec((2, 128), lambda i: (0, 0)),
    ],
    out_shape=[
        jax.ShapeDtypeStruct((2, N, TW), jnp.float32),
        jax.ShapeDtypeStruct((2, 128), jnp.float32),
    ],
)


def _sc_body(t_hbm, m_hbm, sd_hbm, acc_out,
             sdrow, cb0, cb1, cb2, cb3, msg0, msg1, msg2, msg3,
             dsv, mbuf, acc,
             gsem0, gsem1, gsem2, gsem3, ssem0, ssem1, ssem2, ssem3,
             isem0, isem1, isem2, isem3, isem4, isem5, isem6, isem7):
    c = lax.axis_index("c")
    s = lax.axis_index("s")
    iota = lax.broadcasted_iota(jnp.int32, (16,), 0)
    zeros16 = jnp.zeros((16,), jnp.float32)
    c4 = c * 4
    qidx = [jnp.broadcast_to(c4 + j, (16,)) for j in range(4)]
    rotidx = (iota + c4) & 15

    combs = (cb0, cb1, cb2, cb3)
    msgs = (msg0, msg1, msg2, msg3)
    gsems = (gsem0, gsem1, gsem2, gsem3)
    ssems = (ssem0, ssem1, ssem2, ssem3)
    isems = (isem0, isem1, isem2, isem3, isem4, isem5, isem6, isem7)

    def _zrow(r, _):
        for k in range(ACC_W // 16):
            msg0[r, pl.ds(k * 16, 16)] = zeros16
            msg1[r, pl.ds(k * 16, 16)] = zeros16
            msg2[r, pl.ds(k * 16, 16)] = zeros16
            msg3[r, pl.ds(k * 16, 16)] = zeros16
        return 0
    lax.fori_loop(0, CH, _zrow, 0)

    nch = jnp.where(s < 15, RB // ZR, (N - 15 * RB) // ZR)

    def _zacc(k, _):
        r0 = pl.multiple_of(s * RB + k * ZR, ZR)
        pltpu.sync_copy(msg0.at[pl.ds(0, ZR), :], acc.at[pl.ds(r0, ZR), :])
        return 0
    lax.fori_loop(0, nch, _zacc, 0)

    pltpu.sync_copy(m_hbm, mbuf)
    msum = mbuf[0, pl.ds(0, 16)] + mbuf[1, pl.ds(0, 16)]
    m16 = jnp.maximum(msum, msum * ALPHA)
    plsc.subcore_barrier()

    sd0 = (c * 16 + s) * (NIT * SDW)

    def idx_copy(i, b8):
        pltpu.async_copy(sd_hbm.at[pl.ds(sd0 + i * SDW, SDW)],
                         sdrow.at[pl.ds(b8 * SDW, SDW)], isems[b8])

    def idx_wait(i, b8):
        pltpu.make_async_copy(sd_hbm.at[pl.ds(sd0 + i * SDW, SDW)],
                              sdrow.at[pl.ds(b8 * SDW, SDW)], isems[b8]).wait()

    def start_gather(i, b4, b8):
        pltpu.async_copy(
            t_hbm.at[sdrow.at[pl.ds(b8 * SDW, SDW)]], combs[b4], gsems[b4])

    def wait_gather(i, b4, b8):
        pltpu.make_async_copy(
            t_hbm.at[sdrow.at[pl.ds(b8 * SDW, SDW)]], combs[b4],
            gsems[b4]).wait()

    def wait_scatter(b4):
        pltpu.make_async_copy(
            msgs[b4], acc.at[dsv.at[b4]], ssems[b4]).wait()

    for k in range(8):
        idx_copy(k, k)
    for k in range(4):
        idx_wait(k, k)
        start_gather(k, k, k)

    def step(i8, _):
        for b in range(8):
            b4 = b % 4
            i = i8 * 8 + b
            wait_gather(i, b4, b)
            if b < 4:
                @pl.when(i8 >= 1)
                def _():
                    wait_scatter(b4)
            else:
                wait_scatter(b4)

            for k in range(CH // 16):
                dsv[b4, pl.ds(k * 16, 16)] = \
                    sdrow[pl.ds(b * SDW + CH + k * 16, 16)]

            cb = combs[b4]
            msg = msgs[b4]

            @plsc.parallel_loop(0, CH, 1, unroll=4)
            def edge(e):
                va1 = cb[e, pl.ds(64, 16)]
                va2 = cb[CH + e, pl.ds(80, 16)]
                t = va1 + va2
                logit = jnp.maximum(t, t * ALPHA)
                p = jnp.exp(logit - m16)
                for j in range(4):
                    ftb = cb[e, pl.ds(j * 16, 16)]
                    q = jnp.take_along_axis(p, qidx[j], axis=0)
                    msg[e, pl.ds(j * 16, 16)] = ftb * q
                msg[e, pl.ds(64, 16)] = jnp.take_along_axis(p, rotidx, axis=0)

            pltpu.async_copy(msg, acc.at[dsv.at[b4]], ssems[b4], add=True)

            @pl.when(i8 * 8 + b + 4 < NIT)
            def _():
                idx_wait(i + 4, (b + 4) % 8)
                start_gather(i + 4, b4, (b + 4) % 8)

            @pl.when(i8 * 8 + b + 8 < NIT)
            def _():
                idx_copy(i + 8, b)
        return 0
    lax.fori_loop(0, NIT // 8, step, 0)

    for b4 in range(4):
        wait_scatter(b4)
    plsc.subcore_barrier()

    def _wout(k, _):
        r0 = pl.multiple_of(s * RB + k * ZR, ZR)
        pltpu.sync_copy(acc.at[pl.ds(r0, ZR), :],
                        acc_out.at[pl.ds(c * N + r0, ZR), :])
        return 0
    lax.fori_loop(0, nch, _wout, 0)


_sc_edge = pl.kernel(
    _sc_body,
    out_type=jax.ShapeDtypeStruct((2 * N, ACC_W), jnp.float32),
    mesh=plsc.VectorSubcoreMesh(core_axis_name="c", subcore_axis_name="s"),
    compiler_params=pltpu.CompilerParams(needs_layout_passes=False),
    scratch_types=(
        [pltpu.VMEM((8 * SDW,), jnp.int32)]
        + [pltpu.VMEM((2 * CH, TW), jnp.float32) for _ in range(4)]
        + [pltpu.VMEM((CH, ACC_W), jnp.float32) for _ in range(4)]
        + [
            pltpu.VMEM((4, CH), jnp.int32),
            pltpu.VMEM((2, 128), jnp.float32),
            pltpu.VMEM_SHARED((N, ACC_W), jnp.float32),
        ]
        + [pltpu.SemaphoreType.DMA for _ in range(16)]
    ),
)


def _post_body(acc0_ref, acc1_ref, s_ref, out_ref):
    sel = s_ref[...]
    n0 = acc0_ref[:, 0:64]
    z0 = acc0_ref[:, 64:68]
    n1 = acc1_ref[:, 0:64]
    z1 = acc1_ref[:, 64:68]
    r0 = n0 / jnp.dot(z0, sel, preferred_element_type=jnp.float32)
    r1 = n1 / jnp.dot(z1, sel, preferred_element_type=jnp.float32)
    out_ref[...] = jnp.concatenate([r0, r1], axis=1)


_tc_post = pl.pallas_call(
    _post_body,
    grid=(GRID,),
    in_specs=[
        pl.BlockSpec((BR, ACC_W), lambda i: (i, 0)),
        pl.BlockSpec((BR, ACC_W), lambda i: (N // BR + i, 0)),
        pl.BlockSpec((4, 64), lambda i: (0, 0)),
    ],
    out_specs=pl.BlockSpec((BR, HD), lambda i: (i, 0)),
    out_shape=jax.ShapeDtypeStruct((N, HD), jnp.float32),
)


@jax.jit
def kernel(inputs, edge_index, W, attn_l, attn_r):
    wt = W.T
    al = attn_l[:, :, 0]
    ar = attn_r[:, :, 0]
    eye = jnp.eye(H, dtype=jnp.float32)
    a1w = jnp.einsum('hd,hk->hdk', al, eye).reshape(HD, H)
    a2w = jnp.einsum('hd,hk->hdk', ar, eye).reshape(HD, H)
    zpad = jnp.zeros((HD, 128 - 2 * H), jnp.float32)
    a1w = jnp.concatenate([a1w, a1w, zpad], axis=1)
    a2w = jnp.concatenate([a2w, a2w, zpad], axis=1)
    sel = jnp.einsum('jk,d->kjd', jnp.eye(4, dtype=jnp.float32),
                     jnp.ones((D,), jnp.float32)).reshape(4, 64)

    tpack, m = _tc_pre(inputs, wt, a1w, a2w)
    dummy = jnp.zeros((8, TW), jnp.float32).at[:, 64:80].set(-1e30)
    table = jnp.concatenate([tpack.reshape(2 * N, TW), dummy], axis=0)
    src = edge_index[0]
    dst = edge_index[1]
    spad = jnp.full((PAD,), DUMMY, jnp.int32)
    dpad = jnp.zeros((PAD,), jnp.int32)
    srcb = jnp.concatenate([src, spad, src + N, spad]).reshape(2, EPAD // CH, CH)
    dstp = jnp.concatenate([dst, dpad]).reshape(EPAD // CH, CH)
    dst2 = jnp.stack([dstp, dstp])
    sd = jnp.concatenate([srcb, dst2], axis=-1).reshape(-1)

    acc = _sc_edge(table, m, sd)
    out = _tc_post(acc, acc, sel)
    return out.reshape(N, H, D)


# speedup vs baseline: 40.2511x; 1.0148x over previous
import jax
import jax.numpy as jnp
from jax import lax
from jax.experimental import pallas as pl
from jax.experimental.pallas import tpu as pltpu
from jax.experimental.pallas import tpu_sc as plsc

N = 10000
E = 320000
IN_DIM = 128
H = 8
D = 16
HD = H * D
ALPHA = 0.2

BR = 1000
GRID = N // BR
CH = 32
NIT = 640
EP = NIT * CH
EPAD = 16 * EP
PAD = EPAD - E
SDW = 2 * CH
TW = 128
ACC_W = 128
ZR = 16
RB = 624
DUMMY = 2 * N


def _pre_body(x_ref, wt_ref, a1w_ref, a2w_ref, t_ref, m_ref):
    i = pl.program_id(0)
    ft = jnp.dot(x_ref[...], wt_ref[...], preferred_element_type=jnp.float32)
    a1 = jnp.dot(ft, a1w_ref[...], preferred_element_type=jnp.float32)
    a2 = jnp.dot(ft, a2w_ref[...], preferred_element_type=jnp.float32)
    pad = jnp.zeros((BR, 32), jnp.float32)
    t_ref[0] = jnp.concatenate([ft[:, :64], a1[:, :16], a2[:, :16], pad], axis=1)
    t_ref[1] = jnp.concatenate([ft[:, 64:], a1[:, :16], a2[:, :16], pad], axis=1)

    @pl.when(i == 0)
    def _():
        m_ref[...] = jnp.full((2, 128), -jnp.inf, jnp.float32)

    m1 = jnp.max(a1, axis=0, keepdims=True)
    m2 = jnp.max(a2, axis=0, keepdims=True)
    m_ref[0:1, :] = jnp.maximum(m_ref[0:1, :], m1)
    m_ref[1:2, :] = jnp.maximum(m_ref[1:2, :], m2)


_tc_pre = pl.pallas_call(
    _pre_body,
    grid=(GRID,),
    in_specs=[
        pl.BlockSpec((BR, IN_DIM), lambda i: (i, 0)),
        pl.BlockSpec((IN_DIM, HD), lambda i: (0, 0)),
        pl.BlockSpec((HD, 128), lambda i: (0, 0)),
        pl.BlockSpec((HD, 128), lambda i: (0, 0)),
    ],
    out_specs=[
        pl.BlockSpec((2, BR, TW), lambda i: (0, i, 0)),
        pl.BlockSpec((2, 128), lambda i: (0, 0)),
    ],
    out_shape=[
        jax.ShapeDtypeStruct((2, N, TW), jnp.float32),
        jax.ShapeDtypeStruct((2, 128), jnp.float32),
    ],
)


def _sc_body(t_hbm, m_hbm, sd_hbm, acc_out,
             sdrow, cb0, cb1, cb2, cb3, msg0, msg1, msg2, msg3,
             dsv, mbuf, acc,
             gsem0, gsem1, gsem2, gsem3, hsem0, hsem1, hsem2, hsem3,
             ssem0, ssem1, ssem2, ssem3,
             isem0, isem1, isem2, isem3, isem4, isem5, isem6, isem7):
    c = lax.axis_index("c")
    s = lax.axis_index("s")
    iota = lax.broadcasted_iota(jnp.int32, (16,), 0)
    zeros16 = jnp.zeros((16,), jnp.float32)
    c4 = c * 4
    qidx = [jnp.broadcast_to(c4 + j, (16,)) for j in range(4)]
    rotidx = (iota + c4) & 15

    combs = (cb0, cb1, cb2, cb3)
    msgs = (msg0, msg1, msg2, msg3)
    gsems = (gsem0, gsem1, gsem2, gsem3)
    hsems = (hsem0, hsem1, hsem2, hsem3)
    ssems = (ssem0, ssem1, ssem2, ssem3)
    isems = (isem0, isem1, isem2, isem3, isem4, isem5, isem6, isem7)

    def _zrow(r, _):
        for k in range(ACC_W // 16):
            msg0[r, pl.ds(k * 16, 16)] = zeros16
            msg1[r, pl.ds(k * 16, 16)] = zeros16
            msg2[r, pl.ds(k * 16, 16)] = zeros16
            msg3[r, pl.ds(k * 16, 16)] = zeros16
        return 0
    lax.fori_loop(0, CH, _zrow, 0)

    nch = jnp.where(s < 15, RB // ZR, (N - 15 * RB) // ZR)

    def _zacc(k, _):
        r0 = pl.multiple_of(s * RB + k * ZR, ZR)
        pltpu.sync_copy(msg0.at[pl.ds(0, ZR), :], acc.at[pl.ds(r0, ZR), :])
        return 0
    lax.fori_loop(0, nch, _zacc, 0)

    pltpu.sync_copy(m_hbm, mbuf)
    msum = mbuf[0, pl.ds(0, 16)] + mbuf[1, pl.ds(0, 16)]
    m16 = jnp.maximum(msum, msum * ALPHA)
    plsc.subcore_barrier()

    sd0 = (c * 16 + s) * (NIT * SDW)

    def idx_copy(i, b8):
        pltpu.async_copy(sd_hbm.at[pl.ds(sd0 + i * SDW, SDW)],
                         sdrow.at[pl.ds(b8 * SDW, SDW)], isems[b8])

    def idx_wait(i, b8):
        pltpu.make_async_copy(sd_hbm.at[pl.ds(sd0 + i * SDW, SDW)],
                              sdrow.at[pl.ds(b8 * SDW, SDW)], isems[b8]).wait()

    def start_gather(i, b4, b8):
        pltpu.async_copy(
            t_hbm.at[sdrow.at[pl.ds(b8 * SDW, CH)]],
            combs[b4].at[pl.ds(0, CH), :], gsems[b4])
        pltpu.async_copy(
            t_hbm.at[sdrow.at[pl.ds(b8 * SDW + CH, CH)]],
            combs[b4].at[pl.ds(CH, CH), :], hsems[b4])

    def wait_gather(i, b4, b8):
        pltpu.make_async_copy(
            t_hbm.at[sdrow.at[pl.ds(b8 * SDW, CH)]],
            combs[b4].at[pl.ds(0, CH), :], gsems[b4]).wait()
        pltpu.make_async_copy(
            t_hbm.at[sdrow.at[pl.ds(b8 * SDW + CH, CH)]],
            combs[b4].at[pl.ds(CH, CH), :], hsems[b4]).wait()

    def wait_scatter(b4):
        pltpu.make_async_copy(
            msgs[b4], acc.at[dsv.at[b4]], ssems[b4]).wait()

    for k in range(8):
        idx_copy(k, k)
    for k in range(4):
        idx_wait(k, k)
        start_gather(k, k, k)

    def step(i8, _):
        for b in range(8):
            b4 = b % 4
            i = i8 * 8 + b
            wait_gather(i, b4, b)
            if b < 4:
                @pl.when(i8 >= 1)
                def _():
                    wait_scatter(b4)
            else:
                wait_scatter(b4)

            for k in range(CH // 16):
                dsv[b4, pl.ds(k * 16, 16)] = \
                    sdrow[pl.ds(b * SDW + CH + k * 16, 16)]

            cb = combs[b4]
            msg = msgs[b4]

            @plsc.parallel_loop(0, CH, 1, unroll=4)
            def edge(e):
                va1 = cb[e, pl.ds(64, 16)]
                va2 = cb[CH + e, pl.ds(80, 16)]
                t = va1 + va2
                logit = jnp.maximum(t, t * ALPHA)
                p = jnp.exp(logit - m16)
                for j in range(4):
                    ftb = cb[e, pl.ds(j * 16, 16)]
                    q = jnp.take_along_axis(p, qidx[j], axis=0)
                    msg[e, pl.ds(j * 16, 16)] = ftb * q
                msg[e, pl.ds(64, 16)] = jnp.take_along_axis(p, rotidx, axis=0)

            pltpu.async_copy(msg, acc.at[dsv.at[b4]], ssems[b4], add=True)

            @pl.when(i8 * 8 + b + 4 < NIT)
            def _():
                idx_wait(i + 4, (b + 4) % 8)
                start_gather(i + 4, b4, (b + 4) % 8)

            @pl.when(i8 * 8 + b + 8 < NIT)
            def _():
                idx_copy(i + 8, b)
        return 0
    lax.fori_loop(0, NIT // 8, step, 0)

    for b4 in range(4):
        wait_scatter(b4)
    plsc.subcore_barrier()

    def _wout(k, _):
        r0 = pl.multiple_of(s * RB + k * ZR, ZR)
        pltpu.sync_copy(acc.at[pl.ds(r0, ZR), :],
                        acc_out.at[pl.ds(c * N + r0, ZR), :])
        return 0
    lax.fori_loop(0, nch, _wout, 0)


_sc_edge = pl.kernel(
    _sc_body,
    out_type=jax.ShapeDtypeStruct((2 * N, ACC_W), jnp.float32),
    mesh=plsc.VectorSubcoreMesh(core_axis_name="c", subcore_axis_name="s"),
    compiler_params=pltpu.CompilerParams(needs_layout_passes=False),
    scratch_types=(
        [pltpu.VMEM((8 * SDW,), jnp.int32)]
        + [pltpu.VMEM((2 * CH, TW), jnp.float32) for _ in range(4)]
        + [pltpu.VMEM((CH, ACC_W), jnp.float32) for _ in range(4)]
        + [
            pltpu.VMEM((4, CH), jnp.int32),
            pltpu.VMEM((2, 128), jnp.float32),
            pltpu.VMEM_SHARED((N, ACC_W), jnp.float32),
        ]
        + [pltpu.SemaphoreType.DMA for _ in range(20)]
    ),
)


def _post_body(acc0_ref, acc1_ref, s_ref, out_ref):
    sel = s_ref[...]
    n0 = acc0_ref[:, 0:64]
    z0 = acc0_ref[:, 64:68]
    n1 = acc1_ref[:, 0:64]
    z1 = acc1_ref[:, 64:68]
    r0 = n0 / jnp.dot(z0, sel, preferred_element_type=jnp.float32)
    r1 = n1 / jnp.dot(z1, sel, preferred_element_type=jnp.float32)
    out_ref[...] = jnp.concatenate([r0, r1], axis=1)


_tc_post = pl.pallas_call(
    _post_body,
    grid=(GRID,),
    in_specs=[
        pl.BlockSpec((BR, ACC_W), lambda i: (i, 0)),
        pl.BlockSpec((BR, ACC_W), lambda i: (N // BR + i, 0)),
        pl.BlockSpec((4, 64), lambda i: (0, 0)),
    ],
    out_specs=pl.BlockSpec((BR, HD), lambda i: (i, 0)),
    out_shape=jax.ShapeDtypeStruct((N, HD), jnp.float32),
)


@jax.jit
def kernel(inputs, edge_index, W, attn_l, attn_r):
    wt = W.T
    al = attn_l[:, :, 0]
    ar = attn_r[:, :, 0]
    eye = jnp.eye(H, dtype=jnp.float32)
    a1w = jnp.einsum('hd,hk->hdk', al, eye).reshape(HD, H)
    a2w = jnp.einsum('hd,hk->hdk', ar, eye).reshape(HD, H)
    zpad = jnp.zeros((HD, 128 - 2 * H), jnp.float32)
    a1w = jnp.concatenate([a1w, a1w, zpad], axis=1)
    a2w = jnp.concatenate([a2w, a2w, zpad], axis=1)
    sel = jnp.einsum('jk,d->kjd', jnp.eye(4, dtype=jnp.float32),
                     jnp.ones((D,), jnp.float32)).reshape(4, 64)

    tpack, m = _tc_pre(inputs, wt, a1w, a2w)
    dummy = jnp.zeros((8, TW), jnp.float32).at[:, 64:80].set(-1e30)
    table = jnp.concatenate([tpack.reshape(2 * N, TW), dummy], axis=0)
    src = edge_index[0]
    dst = edge_index[1]
    spad = jnp.full((PAD,), DUMMY, jnp.int32)
    dpad = jnp.zeros((PAD,), jnp.int32)
    srcb = jnp.concatenate([src, spad, src + N, spad]).reshape(2, EPAD // CH, CH)
    dstp = jnp.concatenate([dst, dpad]).reshape(EPAD // CH, CH)
    dst2 = jnp.stack([dstp, dstp])
    sd = jnp.concatenate([srcb, dst2], axis=-1).reshape(-1)

    acc = _sc_edge(table, m, sd)
    out = _tc_post(acc, acc, sel)
    return out.reshape(N, H, D)
